# baseline (speedup 1.0000x reference)
import functools

import jax
import jax.numpy as jnp
from jax import lax
from jax.experimental import pallas as pl
from jax.experimental.pallas import tpu as pltpu
from jax.experimental.pallas import tpu_sc as plsc

NUM_VOCAB = 1000000
EMBED_DIM = 64
ROW = 128
HIDDEN_DIM = 256
BATCH = 4096
SEQ = 200

_TBLK = 512
_NBLK = 977
_SPLIT = _TBLK * _NBLK

_INFO = plsc.get_sparse_core_info()
_NC = _INFO.num_cores
_NS = _INFO.num_subcores
_NW = _NC * _NS
_BPW = BATCH // _NW
_SLICES = [(0, 128), (128, 72)]
_IDXPAD = 208
_RAWPAD = 224


def _repack_body(x1_ref, x2_ref, o_ref):
  o_ref[:, 0:EMBED_DIM] = jnp.transpose(x1_ref[...])
  o_ref[:, EMBED_DIM:ROW] = jnp.transpose(x2_ref[...])


def _tc_repack(tableT):
  return pl.pallas_call(
      _repack_body,
      grid=(_NBLK,),
      in_specs=[
          pl.BlockSpec((EMBED_DIM, _TBLK), lambda i: (0, i)),
          pl.BlockSpec((EMBED_DIM, _TBLK), lambda i: (0, _NBLK + i)),
      ],
      out_specs=pl.BlockSpec((_TBLK, ROW), lambda i: (i, 0)),
      out_shape=jax.ShapeDtypeStruct((_SPLIT, ROW), jnp.float32),
  )(tableT, tableT)


def _sc_pool(xf, tp):
  mesh = plsc.VectorSubcoreMesh(core_axis_name="c", subcore_axis_name="s")

  @functools.partial(
      pl.kernel,
      out_type=jax.ShapeDtypeStruct((BATCH, EMBED_DIM), jnp.float32),
      mesh=mesh,
      scratch_types=[
          pltpu.VMEM((_RAWPAD,), jnp.int32),
          pltpu.VMEM((_RAWPAD,), jnp.int32),
          pltpu.VMEM((_IDXPAD,), jnp.int32),
          pltpu.VMEM((_IDXPAD,), jnp.int32),
          pltpu.VMEM((SEQ, ROW), jnp.float32),
          pltpu.VMEM((SEQ, ROW), jnp.float32),
          pltpu.VMEM((_BPW, EMBED_DIM), jnp.float32),
          pltpu.SemaphoreType.DMA,
          pltpu.SemaphoreType.DMA,
      ],
      compiler_params=pltpu.CompilerParams(use_tc_tiling_on_sc=True),
  )
  def k(xf_hbm, tp_hbm, out_hbm,
        raw0, raw1, idx0, idx1, rows0, rows1, out_v, sem0, sem1):
    wid = lax.axis_index("s") * _NC + lax.axis_index("c")
    base = wid * _BPW
    bufs = ((raw0, idx0, rows0, sem0), (raw1, idx1, rows1, sem1))

    def start(c, raw_v, idx_v, rows_v, sem):
      off = (base + c) * SEQ
      pltpu.sync_copy(xf_hbm.at[pl.ds(off, SEQ)], raw_v.at[pl.ds(0, SEQ)])
      for j in range(_IDXPAD // 16):
        v = raw_v[pl.ds(j * 16, 16)]
        idx_v[pl.ds(j * 16, 16)] = jnp.where(v >= _SPLIT, v - _SPLIT, v)
      for o, l in _SLICES:
        pltpu.async_copy(
            tp_hbm.at[idx_v.at[pl.ds(o, l)]], rows_v.at[pl.ds(o, l)], sem
        )

    def finish(c, raw_v, idx_v, rows_v, sem):
      for o, l in _SLICES:
        pltpu.make_async_copy(
            tp_hbm.at[idx_v.at[pl.ds(o, l)]], rows_v.at[pl.ds(o, l)], sem
        ).wait()
      zero = jnp.zeros((16,), jnp.float32)

      @plsc.parallel_loop(0, SEQ, unroll=8, carry=(zero, zero, zero, zero))
      def accs(r, acc):
        v = raw_v[pl.ds(r, 16)]
        h = jnp.where(v[0] >= _SPLIT, EMBED_DIM, 0)
        return tuple(
            acc[g] + rows_v[r, pl.ds(h + g * 16, 16)] for g in range(4)
        )

      for g in range(4):
        out_v[c, pl.ds(g * 16, 16)] = accs[g] * (1.0 / SEQ)

    for b in range(2):
      start(b, *bufs[b])

    def chunk_body(g, carry):
      for b in range(2):
        c = 2 * g + b
        raw_v, idx_v, rows_v, sem = bufs[b]
        finish(c, raw_v, idx_v, rows_v, sem)

        @pl.when(c + 2 < _BPW)
        def _():
          start(c + 2, raw_v, idx_v, rows_v, sem)

      return carry

    lax.fori_loop(0, _BPW // 2, chunk_body, 0)
    pltpu.sync_copy(out_v, out_hbm.at[pl.ds(base, _BPW)])

  return k(xf, tp)


def _mlp_body(h0_ref, w1_ref, b1_ref, w2_ref, b2_ref, o_ref):
  h = h0_ref[...]
  h1 = lax.dot_general(
      h, w1_ref[...], (((1,), (1,)), ((), ())),
      preferred_element_type=jnp.float32,
  )
  h1 = jnp.maximum(h1 + b1_ref[...], 0.0)
  o = jnp.sum(h1 * w2_ref[...], axis=1, keepdims=True) + b2_ref[...]
  o_ref[...] = 1.0 / (1.0 + jnp.exp(-o))


def _tc_mlp(h0, W1, b1, W2, b2):
  nb = 8
  bm = BATCH // nb
  return pl.pallas_call(
      _mlp_body,
      grid=(nb,),
      in_specs=[
          pl.BlockSpec((bm, EMBED_DIM), lambda i: (i, 0)),
          pl.BlockSpec((HIDDEN_DIM, EMBED_DIM), lambda i: (0, 0)),
          pl.BlockSpec((1, HIDDEN_DIM), lambda i: (0, 0)),
          pl.BlockSpec((1, HIDDEN_DIM), lambda i: (0, 0)),
          pl.BlockSpec((1, 1), lambda i: (0, 0)),
      ],
      out_specs=pl.BlockSpec((bm, 1), lambda i: (i, 0)),
      out_shape=jax.ShapeDtypeStruct((BATCH, 1), jnp.float32),
  )(h0, W1, b1, W2, b2)


@jax.jit
def kernel(x, table, W1, b1, W2, b2):
  xf = jnp.reshape(x, (BATCH * SEQ,))
  tp = _tc_repack(jnp.transpose(table))
  h0 = _sc_pool(xf, tp)
  out = _tc_mlp(h0, W1, b1.reshape(1, HIDDEN_DIM), W2, b2.reshape(1, 1))
  return jnp.squeeze(out, axis=1)


# speedup vs baseline: 1.8622x; 1.8622x over previous
import functools

import jax
import jax.numpy as jnp
from jax import lax
from jax.experimental import pallas as pl
from jax.experimental.pallas import tpu as pltpu
from jax.experimental.pallas import tpu_sc as plsc

NUM_VOCAB = 1000000
EMBED_DIM = 64
ROW = 128
HIDDEN_DIM = 256
BATCH = 4096
SEQ = 200

_TBLK = 4096
_NBLK = 123
_SPLIT = _TBLK * _NBLK

_INFO = plsc.get_sparse_core_info()
_NC = _INFO.num_cores
_NS = _INFO.num_subcores
_NW = _NC * _NS
_BPW = BATCH // _NW
_SLICES = [(0, 128), (128, 72)]
_IDXPAD = 208
_RAWPAD = 224


def _repack_body(x1_ref, x2_ref, o_ref):
  eye = jnp.asarray(
      lax.broadcasted_iota(jnp.int32, (EMBED_DIM, EMBED_DIM), 0)
      == lax.broadcasted_iota(jnp.int32, (EMBED_DIM, EMBED_DIM), 1),
      jnp.float32,
  )
  dims = (((0,), (0,)), ((), ()))
  o_ref[:, 0:EMBED_DIM] = lax.dot_general(
      x1_ref[...], eye, dims, preferred_element_type=jnp.float32
  )
  o_ref[:, EMBED_DIM:ROW] = lax.dot_general(
      x2_ref[...], eye, dims, preferred_element_type=jnp.float32
  )


def _tc_repack(tableT):
  return pl.pallas_call(
      _repack_body,
      grid=(_NBLK,),
      in_specs=[
          pl.BlockSpec((EMBED_DIM, _TBLK), lambda i: (0, i)),
          pl.BlockSpec(
              (EMBED_DIM, _TBLK),
              lambda i: (0, jnp.minimum(_NBLK + i, NUM_VOCAB // _TBLK)),
          ),
      ],
      out_specs=pl.BlockSpec((_TBLK, ROW), lambda i: (i, 0)),
      out_shape=jax.ShapeDtypeStruct((_SPLIT, ROW), jnp.float32),
  )(tableT, tableT)


def _sc_pool(xf, tp):
  mesh = plsc.VectorSubcoreMesh(core_axis_name="c", subcore_axis_name="s")

  @functools.partial(
      pl.kernel,
      out_type=jax.ShapeDtypeStruct((BATCH, EMBED_DIM), jnp.float32),
      mesh=mesh,
      scratch_types=[
          pltpu.VMEM((_RAWPAD,), jnp.int32),
          pltpu.VMEM((_RAWPAD,), jnp.int32),
          pltpu.VMEM((_IDXPAD,), jnp.int32),
          pltpu.VMEM((_IDXPAD,), jnp.int32),
          pltpu.VMEM((SEQ, ROW), jnp.float32),
          pltpu.VMEM((SEQ, ROW), jnp.float32),
          pltpu.VMEM((_BPW, EMBED_DIM), jnp.float32),
          pltpu.SemaphoreType.DMA,
          pltpu.SemaphoreType.DMA,
      ],
      compiler_params=pltpu.CompilerParams(use_tc_tiling_on_sc=True),
  )
  def k(xf_hbm, tp_hbm, out_hbm,
        raw0, raw1, idx0, idx1, rows0, rows1, out_v, sem0, sem1):
    wid = lax.axis_index("s") * _NC + lax.axis_index("c")
    base = wid * _BPW
    bufs = ((raw0, idx0, rows0, sem0), (raw1, idx1, rows1, sem1))

    def start(c, raw_v, idx_v, rows_v, sem):
      off = (base + c) * SEQ
      pltpu.sync_copy(xf_hbm.at[pl.ds(off, SEQ)], raw_v.at[pl.ds(0, SEQ)])
      for j in range(_IDXPAD // 16):
        v = raw_v[pl.ds(j * 16, 16)]
        idx_v[pl.ds(j * 16, 16)] = jnp.where(v >= _SPLIT, v - _SPLIT, v)
      for o, l in _SLICES:
        pltpu.async_copy(
            tp_hbm.at[idx_v.at[pl.ds(o, l)]], rows_v.at[pl.ds(o, l)], sem
        )

    def finish(c, raw_v, idx_v, rows_v, sem):
      for o, l in _SLICES:
        pltpu.make_async_copy(
            tp_hbm.at[idx_v.at[pl.ds(o, l)]], rows_v.at[pl.ds(o, l)], sem
        ).wait()
      zero = jnp.zeros((16,), jnp.float32)

      @plsc.parallel_loop(0, SEQ, unroll=8, carry=(zero, zero, zero, zero))
      def accs(r, acc):
        v = raw_v[pl.ds(r, 16)]
        h = jnp.where(v[0] >= _SPLIT, EMBED_DIM, 0)
        return tuple(
            acc[g] + rows_v[r, pl.ds(h + g * 16, 16)] for g in range(4)
        )

      for g in range(4):
        out_v[c, pl.ds(g * 16, 16)] = accs[g] * (1.0 / SEQ)

    for b in range(2):
      start(b, *bufs[b])

    def chunk_body(g, carry):
      for b in range(2):
        c = 2 * g + b
        raw_v, idx_v, rows_v, sem = bufs[b]
        finish(c, raw_v, idx_v, rows_v, sem)

        @pl.when(c + 2 < _BPW)
        def _():
          start(c + 2, raw_v, idx_v, rows_v, sem)

      return carry

    lax.fori_loop(0, _BPW // 2, chunk_body, 0)
    pltpu.sync_copy(out_v, out_hbm.at[pl.ds(base, _BPW)])

  return k(xf, tp)


def _mlp_body(h0_ref, w1_ref, b1_ref, w2_ref, b2_ref, o_ref):
  h = h0_ref[...]
  h1 = lax.dot_general(
      h, w1_ref[...], (((1,), (1,)), ((), ())),
      preferred_element_type=jnp.float32,
  )
  h1 = jnp.maximum(h1 + b1_ref[...], 0.0)
  o = jnp.sum(h1 * w2_ref[...], axis=1, keepdims=True) + b2_ref[...]
  o_ref[...] = 1.0 / (1.0 + jnp.exp(-o))


def _tc_mlp(h0, W1, b1, W2, b2):
  nb = 8
  bm = BATCH // nb
  return pl.pallas_call(
      _mlp_body,
      grid=(nb,),
      in_specs=[
          pl.BlockSpec((bm, EMBED_DIM), lambda i: (i, 0)),
          pl.BlockSpec((HIDDEN_DIM, EMBED_DIM), lambda i: (0, 0)),
          pl.BlockSpec((1, HIDDEN_DIM), lambda i: (0, 0)),
          pl.BlockSpec((1, HIDDEN_DIM), lambda i: (0, 0)),
          pl.BlockSpec((1, 1), lambda i: (0, 0)),
      ],
      out_specs=pl.BlockSpec((bm, 1), lambda i: (i, 0)),
      out_shape=jax.ShapeDtypeStruct((BATCH, 1), jnp.float32),
  )(h0, W1, b1, W2, b2)


@jax.jit
def kernel(x, table, W1, b1, W2, b2):
  xf = jnp.reshape(x, (BATCH * SEQ,))
  tp = _tc_repack(jnp.transpose(table))
  h0 = _sc_pool(xf, tp)
  out = _tc_mlp(h0, W1, b1.reshape(1, HIDDEN_DIM), W2, b2.reshape(1, 1))
  return jnp.squeeze(out, axis=1)


# speedup vs baseline: 2.1094x; 1.1328x over previous
import functools

import jax
import jax.numpy as jnp
from jax import lax
from jax.experimental import pallas as pl
from jax.experimental.pallas import tpu as pltpu
from jax.experimental.pallas import tpu_sc as plsc

NUM_VOCAB = 1000000
EMBED_DIM = 64
ROW = 128
HIDDEN_DIM = 256
BATCH = 4096
SEQ = 200

_TBLK = 8192
_NBLK = 62
_SPLIT = _TBLK * _NBLK

_INFO = plsc.get_sparse_core_info()
_NC = _INFO.num_cores
_NS = _INFO.num_subcores
_NW = _NC * _NS
_BPW = BATCH // _NW
_IPW = _BPW * SEQ
_SLICES = [(0, 128), (128, 72)]


def _repack_body(x1_ref, x2_ref, o_ref):
  eye = jnp.asarray(
      lax.broadcasted_iota(jnp.int32, (EMBED_DIM, EMBED_DIM), 0)
      == lax.broadcasted_iota(jnp.int32, (EMBED_DIM, EMBED_DIM), 1),
      jnp.float32,
  )
  dims = (((0,), (0,)), ((), ()))
  o_ref[:, 0:EMBED_DIM] = lax.dot_general(
      x1_ref[...], eye, dims, preferred_element_type=jnp.float32
  )
  o_ref[:, EMBED_DIM:ROW] = lax.dot_general(
      x2_ref[...], eye, dims, preferred_element_type=jnp.float32
  )


def _tc_repack(tableT):
  return pl.pallas_call(
      _repack_body,
      grid=(_NBLK,),
      in_specs=[
          pl.BlockSpec((EMBED_DIM, _TBLK), lambda i: (0, i)),
          pl.BlockSpec(
              (EMBED_DIM, _TBLK),
              lambda i: (0, jnp.minimum(_NBLK + i, NUM_VOCAB // _TBLK)),
          ),
      ],
      out_specs=pl.BlockSpec((_TBLK, ROW), lambda i: (i, 0)),
      out_shape=jax.ShapeDtypeStruct((_SPLIT, ROW), jnp.float32),
  )(tableT, tableT)


def _sc_pool(xf, tp):
  mesh = plsc.VectorSubcoreMesh(core_axis_name="c", subcore_axis_name="s")

  @functools.partial(
      pl.kernel,
      out_type=jax.ShapeDtypeStruct((BATCH, EMBED_DIM), jnp.float32),
      mesh=mesh,
      scratch_types=[
          pltpu.VMEM((_IPW + 16,), jnp.int32),
          pltpu.VMEM((_IPW,), jnp.int32),
          pltpu.VMEM((SEQ, ROW), jnp.float32),
          pltpu.VMEM((SEQ, ROW), jnp.float32),
          pltpu.VMEM((_BPW, EMBED_DIM), jnp.float32),
          pltpu.SemaphoreType.DMA,
          pltpu.SemaphoreType.DMA,
      ],
      compiler_params=pltpu.CompilerParams(use_tc_tiling_on_sc=True),
  )
  def k(xf_hbm, tp_hbm, out_hbm, raw_v, idx_v, rows0, rows1, out_v,
        sem0, sem1):
    wid = lax.axis_index("s") * _NC + lax.axis_index("c")
    base = wid * _BPW
    bufs = ((rows0, sem0), (rows1, sem1))

    pltpu.sync_copy(xf_hbm.at[pl.ds(base * SEQ, _IPW)],
                    raw_v.at[pl.ds(0, _IPW)])

    @plsc.parallel_loop(0, _IPW // 16, unroll=8)
    def _(j):
      v = raw_v[pl.ds(j * 16, 16)]
      idx_v[pl.ds(j * 16, 16)] = jnp.where(v >= _SPLIT, v - _SPLIT, v)

    def start(c, rows_v, sem):
      for o, l in _SLICES:
        pltpu.async_copy(
            tp_hbm.at[idx_v.at[pl.ds(c * SEQ + o, l)]],
            rows_v.at[pl.ds(o, l)], sem,
        )

    def finish(c, rows_v, sem):
      for o, l in _SLICES:
        pltpu.make_async_copy(
            tp_hbm.at[idx_v.at[pl.ds(c * SEQ + o, l)]],
            rows_v.at[pl.ds(o, l)], sem,
        ).wait()
      zero = jnp.zeros((16,), jnp.float32)

      @plsc.parallel_loop(0, SEQ, unroll=8, carry=(zero, zero, zero, zero))
      def accs(r, acc):
        v = raw_v[pl.ds(c * SEQ + r, 16)]
        h = jnp.where(v[0] >= _SPLIT, EMBED_DIM, 0)
        return tuple(
            acc[g] + rows_v[r, pl.ds(h + g * 16, 16)] for g in range(4)
        )

      for g in range(4):
        out_v[c, pl.ds(g * 16, 16)] = accs[g] * (1.0 / SEQ)

    for b in range(2):
      start(b, *bufs[b])

    def chunk_body(g, carry):
      for b in range(2):
        c = 2 * g + b
        rows_v, sem = bufs[b]
        finish(c, rows_v, sem)

        @pl.when(c + 2 < _BPW)
        def _():
          start(c + 2, rows_v, sem)

      return carry

    lax.fori_loop(0, _BPW // 2, chunk_body, 0)
    pltpu.sync_copy(out_v, out_hbm.at[pl.ds(base, _BPW)])

  return k(xf, tp)


def _mlp_body(h0_ref, w1_ref, b1_ref, w2_ref, b2_ref, o_ref):
  h = h0_ref[...]
  h1 = lax.dot_general(
      h, w1_ref[...], (((1,), (1,)), ((), ())),
      preferred_element_type=jnp.float32,
  )
  h1 = jnp.maximum(h1 + b1_ref[...], 0.0)
  o = jnp.sum(h1 * w2_ref[...], axis=1, keepdims=True) + b2_ref[...]
  o_ref[...] = 1.0 / (1.0 + jnp.exp(-o))


def _tc_mlp(h0, W1, b1, W2, b2):
  nb = 8
  bm = BATCH // nb
  return pl.pallas_call(
      _mlp_body,
      grid=(nb,),
      in_specs=[
          pl.BlockSpec((bm, EMBED_DIM), lambda i: (i, 0)),
          pl.BlockSpec((HIDDEN_DIM, EMBED_DIM), lambda i: (0, 0)),
          pl.BlockSpec((1, HIDDEN_DIM), lambda i: (0, 0)),
          pl.BlockSpec((1, HIDDEN_DIM), lambda i: (0, 0)),
          pl.BlockSpec((1, 1), lambda i: (0, 0)),
      ],
      out_specs=pl.BlockSpec((bm, 1), lambda i: (i, 0)),
      out_shape=jax.ShapeDtypeStruct((BATCH, 1), jnp.float32),
  )(h0, W1, b1, W2, b2)


@jax.jit
def kernel(x, table, W1, b1, W2, b2):
  xf = jnp.reshape(x, (BATCH * SEQ,))
  tp = _tc_repack(jnp.transpose(table))
  h0 = _sc_pool(xf, tp)
  out = _tc_mlp(h0, W1, b1.reshape(1, HIDDEN_DIM), W2, b2.reshape(1, 1))
  return jnp.squeeze(out, axis=1)
